# baseline (speedup 1.0000x reference)
import functools

import jax
import jax.numpy as jnp
from jax import lax
from jax.experimental import pallas as pl
from jax.experimental.pallas import tpu as pltpu
from jax.experimental.pallas import tpu_sc as plsc

FACTOR = 16
EMBED = 64
NC = 2
NS = 16
NW = NC * NS

KJ = 8
CHUNK = KJ * 128
SKEW = CHUNK + 1


C2 = 1600


def _make_table_transpose(v_rows):
    n_chunks = v_rows // C2
    per_w = (n_chunks + NW - 1) // NW
    mesh = plsc.VectorSubcoreMesh(core_axis_name="c", subcore_axis_name="s")

    @functools.partial(
        pl.kernel,
        mesh=mesh,
        out_type=jax.ShapeDtypeStruct((v_rows, FACTOR), jnp.float32),
        scratch_types=[
            pltpu.VMEM((FACTOR, C2), jnp.float32),
            pltpu.VMEM((C2, 17), jnp.float32),
        ],
        compiler_params=pltpu.CompilerParams(
            use_tc_tiling_on_sc=False, needs_layout_passes=False),
    )
    def table_t(t16_h, out_h, colbuf, rowbuf):
        wid = lax.axis_index("s") * NC + lax.axis_index("c")
        riota = lax.iota(jnp.int32, 16)
        zeros = jnp.zeros((16,), jnp.int32)

        def chunk_body(ci, carry):
            c = wid + ci * NW

            @pl.when(c < n_chunks)
            def _():
                v0 = c * C2
                pltpu.sync_copy(t16_h.at[:, pl.ds(v0, C2)], colbuf)

                def t_body(t, carry2):
                    ridx = riota + t * 16
                    for f in range(16):
                        val = colbuf[f, pl.ds(t * 16, 16)]
                        plsc.store_scatter(rowbuf, [ridx, zeros + f], val)
                    return carry2

                lax.fori_loop(0, C2 // 16, t_body, 0)
                pltpu.sync_copy(
                    rowbuf.at[:, pl.ds(0, FACTOR)],
                    out_h.at[pl.ds(v0, C2)],
                )
            return carry

        lax.fori_loop(0, per_w, chunk_body, 0)

    return table_t


def _make_gather(n_l, n_b):
    n_chunks = (n_l * n_b) // CHUNK
    chunks_per_w = n_chunks // NW
    cb_per_l = n_b // CHUNK
    mesh = plsc.VectorSubcoreMesh(core_axis_name="c", subcore_axis_name="s")

    @functools.partial(
        pl.kernel,
        mesh=mesh,
        out_type=jax.ShapeDtypeStruct((n_l * FACTOR, n_b), jnp.float32),
        scratch_types=[
            pltpu.VMEM((KJ, 128), jnp.int32),
            pltpu.VMEM((CHUNK, FACTOR), jnp.float32),
            pltpu.VMEM((FACTOR, SKEW), jnp.float32),
            pltpu.SemaphoreType.DMA,
        ],
        compiler_params=pltpu.CompilerParams(
            use_tc_tiling_on_sc=False, needs_layout_passes=False),
    )
    def gather(table_h, idx_h, out_h, idx_v, rows_v, trans_v, sem):
        wid = lax.axis_index("s") * NC + lax.axis_index("c")
        base = wid * chunks_per_w
        fidx = lax.iota(jnp.int32, 16)
        zeros = jnp.zeros((16,), jnp.int32)

        def chunk_body(ci, carry):
            c = base + ci
            l = c // cb_per_l
            cb = c % cb_per_l
            pltpu.sync_copy(idx_h.at[pl.ds(c * KJ, KJ)], idx_v)
            copies = [
                pltpu.async_copy(
                    table_h.at[idx_v.at[jj]],
                    rows_v.at[pl.ds(jj * 128, 128)],
                    sem,
                )
                for jj in range(KJ)
            ]
            for cp in copies:
                cp.wait()

            def t_body(t, carry2):
                for jj in range(16):
                    j = t * 16 + jj
                    val = rows_v[j, :]
                    plsc.store_scatter(trans_v, [fidx, zeros + j], val)
                return carry2

            lax.fori_loop(0, CHUNK // 16, t_body, 0)

            pltpu.sync_copy(
                trans_v.at[:, pl.ds(0, CHUNK)],
                out_h.at[pl.ds(l * FACTOR, FACTOR), pl.ds(cb * CHUNK, CHUNK)],
            )
            return carry

        lax.fori_loop(0, chunks_per_w, chunk_body, 0)

    return gather


C = 16384


def _proj_body(ft_ref, w_ref, o_ref):
    w = w_ref[...]
    for cb in range(C // 128):
        rhs = ft_ref[0, :, cb, :]
        o_ref[0, :, pl.ds(cb * 128, 128)] = lax.dot_general(
            w, rhs, (((1,), (0,)), ((), ())),
            preferred_element_type=jnp.float32)


def _project(ft4, w, n_l, n_b):
    return pl.pallas_call(
        _proj_body,
        grid=(n_l,),
        in_specs=[
            pl.BlockSpec((1, FACTOR, n_b // 128, 128), lambda l: (l, 0, 0, 0)),
            pl.BlockSpec((EMBED, FACTOR), lambda l: (0, 0)),
        ],
        out_specs=pl.BlockSpec((1, EMBED, C), lambda l: (l, 0, 0)),
        out_shape=jax.ShapeDtypeStruct((n_l, EMBED, n_b), jnp.float32),
    )(ft4, w)


def kernel(x, table, W):
    b, l = x.shape
    n = b * l
    idx2 = x.T.reshape(n // 128, 128).astype(jnp.int32)
    tlin = _make_table_transpose(table.shape[0])(table.T)
    ft = _make_gather(l, b)(tlin, idx2)
    ft4 = ft.reshape(l, FACTOR, b // 128, 128)
    o3 = _project(ft4, W, l, b)
    return jnp.transpose(o3, (2, 0, 1))


# speedup vs baseline: 2.7390x; 2.7390x over previous
import functools

import jax
import jax.numpy as jnp
from jax import lax
from jax.experimental import pallas as pl
from jax.experimental.pallas import tpu as pltpu
from jax.experimental.pallas import tpu_sc as plsc

FACTOR = 16
EMBED = 64
NC = 2
NS = 16
NW = NC * NS

KJ = 8
CHUNK = KJ * 128
SKEW = CHUNK + 1


def _make_gather(n_l, n_b):
    n_chunks = (n_l * n_b) // CHUNK
    chunks_per_w = n_chunks // NW
    cb_per_l = n_b // CHUNK
    mesh = plsc.VectorSubcoreMesh(core_axis_name="c", subcore_axis_name="s")

    @functools.partial(
        pl.kernel,
        mesh=mesh,
        out_type=jax.ShapeDtypeStruct((n_l * FACTOR, n_b), jnp.float32),
        scratch_types=[
            pltpu.VMEM((2, KJ, 128), jnp.int32),
            pltpu.VMEM((2, CHUNK, FACTOR), jnp.float32),
            pltpu.VMEM((FACTOR, SKEW), jnp.float32),
            pltpu.SemaphoreType.DMA,
            pltpu.SemaphoreType.DMA,
        ],
        compiler_params=pltpu.CompilerParams(
            use_tc_tiling_on_sc=False, needs_layout_passes=False),
    )
    def gather(table_h, idx_h, out_h, idx_v, rows_v, trans_v, sem0, sem1):
        wid = lax.axis_index("s") * NC + lax.axis_index("c")
        base = wid * chunks_per_w
        fidx = lax.iota(jnp.int32, 16)
        zeros = jnp.zeros((16,), jnp.int32)
        sems = (sem0, sem1)

        def fire(c, buf, sem):
            pltpu.sync_copy(idx_h.at[pl.ds(c * KJ, KJ)], idx_v.at[buf])
            for jj in range(KJ):
                pltpu.async_copy(
                    table_h.at[idx_v.at[buf, jj]],
                    rows_v.at[buf, pl.ds(jj * 128, 128)],
                    sem,
                )

        def drain(c, buf, sem):
            for jj in range(KJ):
                pltpu.make_async_copy(
                    table_h.at[idx_v.at[buf, jj]],
                    rows_v.at[buf, pl.ds(jj * 128, 128)],
                    sem,
                ).wait()

        def process(c, buf):
            def t_body(t, carry2):
                for jj in range(16):
                    j = t * 16 + jj
                    val = rows_v[buf, j, :]
                    plsc.store_scatter(trans_v, [fidx, zeros + j], val)
                return carry2

            lax.fori_loop(0, CHUNK // 16, t_body, 0)
            l = c // cb_per_l
            cb = c % cb_per_l
            pltpu.sync_copy(
                trans_v.at[:, pl.ds(0, CHUNK)],
                out_h.at[pl.ds(l * FACTOR, FACTOR), pl.ds(cb * CHUNK, CHUNK)],
            )

        n_pairs = chunks_per_w // 2
        fire(base, 0, sems[0])

        def pair_body(p, carry):
            c0 = base + 2 * p
            fire(c0 + 1, 1, sems[1])
            drain(c0, 0, sems[0])
            process(c0, 0)

            @pl.when(2 * p + 2 < chunks_per_w)
            def _():
                fire(c0 + 2, 0, sems[0])

            drain(c0 + 1, 1, sems[1])
            process(c0 + 1, 1)
            return carry

        lax.fori_loop(0, n_pairs, pair_body, 0)
        if chunks_per_w % 2:
            c_last = base + chunks_per_w - 1
            drain(c_last, 0, sems[0])
            process(c_last, 0)

    return gather


C = 16384


def _proj_body(ft_ref, w_ref, o_ref):
    w = w_ref[...]
    for cb in range(C // 128):
        rhs = ft_ref[0, :, cb, :]
        o_ref[0, :, pl.ds(cb * 128, 128)] = lax.dot_general(
            w, rhs, (((1,), (0,)), ((), ())),
            preferred_element_type=jnp.float32)


def _project(ft4, w, n_l, n_b):
    return pl.pallas_call(
        _proj_body,
        grid=(n_l,),
        in_specs=[
            pl.BlockSpec((1, FACTOR, n_b // 128, 128), lambda l: (l, 0, 0, 0)),
            pl.BlockSpec((EMBED, FACTOR), lambda l: (0, 0)),
        ],
        out_specs=pl.BlockSpec((1, EMBED, C), lambda l: (l, 0, 0)),
        out_shape=jax.ShapeDtypeStruct((n_l, EMBED, n_b), jnp.float32),
    )(ft4, w)


def kernel(x, table, W):
    b, l = x.shape
    n = b * l
    idx2 = x.T.reshape(n // 128, 128).astype(jnp.int32)
    ft = _make_gather(l, b)(table, idx2)
    ft4 = ft.reshape(l, FACTOR, b // 128, 128)
    o3 = _project(ft4, W, l, b)
    return jnp.transpose(o3, (2, 0, 1))
